# baseline (speedup 1.0000x reference)
import jax
import jax.numpy as jnp
from jax.experimental import pallas as pl
from jax.experimental.pallas import tpu as pltpu


def _vq_body(x_ref, em2_ref, ehi_ref, elo_ref, e2_ref, x2_ref,
             q_ref, ind_ref, diff_ref, perp_ref,
             counts_ref, dacc_ref):
    b = pl.program_id(0)
    nb = pl.num_programs(0)
    K, D = em2_ref.shape
    UB = x_ref.shape[0]
    M = x_ref.shape[2]
    KF = float(K)

    iota_f = jax.lax.broadcasted_iota(jnp.int32, (K, M), 0).astype(jnp.float32)
    ones_row = jnp.ones((1, M), jnp.bfloat16)
    cnt_total = None
    sq_total = None
    for j in range(UB):
        xb = x_ref[j]
        crossm2 = jax.lax.dot_general(
            em2_ref[...], xb.astype(jnp.bfloat16),
            (((1,), (0,)), ((), ())), preferred_element_type=jnp.float32)

        dist = (e2_ref[...] + x2_ref[j]) + crossm2

        mn = jnp.min(dist, axis=0, keepdims=True)
        cand = jnp.where(dist == mn, iota_f, KF)
        idxf = jnp.min(cand, axis=0, keepdims=True)
        ind_ref[j] = idxf.astype(jnp.int32)

        oh = jnp.where(iota_f == idxf, 1.0, 0.0).astype(jnp.bfloat16)

        q = jax.lax.dot_general(ehi_ref[...], oh, (((0,), (0,)), ((), ())),
                                preferred_element_type=jnp.float32)
        qlo = jax.lax.dot_general(elo_ref[...], oh, (((0,), (0,)), ((), ())),
                                  preferred_element_type=jnp.float32)
        q = q + qlo * (1.0 / 256.0)
        q_ref[j] = q

        cnt = jax.lax.dot_general(ones_row, oh, (((1,), (1,)), ((), ())),
                                  preferred_element_type=jnp.float32)
        s = q - xb
        sq = jnp.sum(s * s, axis=(0, 1), keepdims=True)
        cnt_total = cnt if cnt_total is None else cnt_total + cnt
        sq_total = sq if sq_total is None else sq_total + sq

    @pl.when(b == 0)
    def _init():
        counts_ref[...] = jnp.zeros_like(counts_ref)
        dacc_ref[...] = jnp.zeros_like(dacc_ref)

    counts_ref[...] += cnt_total
    dacc_ref[...] += sq_total

    @pl.when(b == nb - 1)
    def _fini():
        total = nb * UB * M * D
        diff_ref[...] = dacc_ref[...] / total
        avg = counts_ref[...] * (1.0 / (nb * UB * M))
        ent = jnp.sum(avg * jnp.log(avg + 1e-10), axis=(0, 1), keepdims=True)
        perp_ref[...] = jnp.exp(-ent)


def kernel(x, embed):
    B, C, H, W = x.shape
    N, K, D = embed.shape
    M = H * W

    xr = jnp.transpose(x.reshape(B, N, D, H, W), (1, 0, 3, 4, 2))
    x_flat = jax.lax.stop_gradient(xr).reshape(N, -1, D)
    x2 = jnp.sum(x_flat ** 2, axis=2)
    e2 = jnp.sum(embed ** 2, axis=2)

    xv = x.reshape(B, D, M)
    emb = embed.reshape(K, D)
    em2 = (-2.0 * emb).astype(jnp.bfloat16)
    ebits = jax.lax.bitcast_convert_type(emb, jnp.uint32)
    ehi32 = jax.lax.bitcast_convert_type(
        ebits & jnp.uint32(0xFFFF0000), jnp.float32)
    ehi = ehi32.astype(jnp.bfloat16)
    elo = ((emb - ehi32) * 256.0).astype(jnp.bfloat16)
    x2r = x2.reshape(B, 1, M)
    e2c = e2.reshape(K, 1)

    out_shape = [
        jax.ShapeDtypeStruct((B, D, M), jnp.float32),
        jax.ShapeDtypeStruct((B, 1, M), jnp.int32),
        jax.ShapeDtypeStruct((1, 1), jnp.float32),
        jax.ShapeDtypeStruct((1, 1), jnp.float32),
    ]
    UB = 4
    grid = (B // UB,)
    q, ind, diffo, perpo = pl.pallas_call(
        _vq_body,
        grid=grid,
        in_specs=[
            pl.BlockSpec((UB, D, M), lambda b: (b, 0, 0)),
            pl.BlockSpec((K, D), lambda b: (0, 0)),
            pl.BlockSpec((K, D), lambda b: (0, 0)),
            pl.BlockSpec((K, D), lambda b: (0, 0)),
            pl.BlockSpec((K, 1), lambda b: (0, 0)),
            pl.BlockSpec((UB, 1, M), lambda b: (b, 0, 0)),
        ],
        out_specs=[
            pl.BlockSpec((UB, D, M), lambda b: (b, 0, 0)),
            pl.BlockSpec((UB, 1, M), lambda b: (b, 0, 0)),
            pl.BlockSpec((1, 1), lambda b: (0, 0)),
            pl.BlockSpec((1, 1), lambda b: (0, 0)),
        ],
        out_shape=out_shape,
        scratch_shapes=[
            pltpu.VMEM((1, K), jnp.float32),
            pltpu.VMEM((1, 1), jnp.float32),
        ],
    )(xv, em2, ehi, elo, e2c, x2r)

    quantized = q.reshape(B, C, H, W)
    embed_ind = ind.reshape(B, N, H, W)
    diff = diffo.reshape(())
    perplexity = perpo.reshape(N)
    return (quantized, diff, embed_ind, perplexity)


# speedup vs baseline: 1.0547x; 1.0547x over previous
import jax
import jax.numpy as jnp
from jax.experimental import pallas as pl
from jax.experimental.pallas import tpu as pltpu


def _vq_body(x_ref, em2_ref, ecat_ref, e2_ref, x2_ref,
             q_ref, ind_ref, diff_ref, perp_ref,
             counts_ref, dacc_ref):
    b = pl.program_id(0)
    nb = pl.num_programs(0)
    K, D = em2_ref.shape
    UB = x_ref.shape[0]
    M = x_ref.shape[2]
    KF = float(K)

    iota_f = jax.lax.broadcasted_iota(jnp.int32, (K, M), 0).astype(jnp.float32)
    ones_row = jnp.ones((1, M), jnp.bfloat16)
    cnt_total = None
    sq_total = None
    for j in range(UB):
        xb = x_ref[j]
        crossm2 = jax.lax.dot_general(
            em2_ref[...], xb.astype(jnp.bfloat16),
            (((1,), (0,)), ((), ())), preferred_element_type=jnp.float32)

        dist = (e2_ref[...] + x2_ref[j]) + crossm2

        mn = jnp.min(dist, axis=0, keepdims=True)
        cand = jnp.where(dist == mn, iota_f, KF)
        idxf = jnp.min(cand, axis=0, keepdims=True)
        ind_ref[j] = idxf.astype(jnp.int32)

        oh = jnp.where(iota_f == idxf, 1.0, 0.0).astype(jnp.bfloat16)

        qq = jax.lax.dot_general(ecat_ref[...], oh, (((0,), (0,)), ((), ())),
                                 preferred_element_type=jnp.float32)
        q = qq[:D] + qq[D:] * (1.0 / 256.0)
        q_ref[j] = q

        cnt = jax.lax.dot_general(ones_row, oh, (((1,), (1,)), ((), ())),
                                  preferred_element_type=jnp.float32)
        s = q - xb
        sq = jnp.sum(s * s, axis=(0, 1), keepdims=True)
        cnt_total = cnt if cnt_total is None else cnt_total + cnt
        sq_total = sq if sq_total is None else sq_total + sq

    @pl.when(b == 0)
    def _init():
        counts_ref[...] = jnp.zeros_like(counts_ref)
        dacc_ref[...] = jnp.zeros_like(dacc_ref)

    counts_ref[...] += cnt_total
    dacc_ref[...] += sq_total

    @pl.when(b == nb - 1)
    def _fini():
        total = nb * UB * M * D
        diff_ref[...] = dacc_ref[...] / total
        avg = counts_ref[...] * (1.0 / (nb * UB * M))
        ent = jnp.sum(avg * jnp.log(avg + 1e-10), axis=(0, 1), keepdims=True)
        perp_ref[...] = jnp.exp(-ent)


def kernel(x, embed):
    B, C, H, W = x.shape
    N, K, D = embed.shape
    M = H * W

    xr = jnp.transpose(x.reshape(B, N, D, H, W), (1, 0, 3, 4, 2))
    x_flat = jax.lax.stop_gradient(xr).reshape(N, -1, D)
    x2 = jnp.sum(x_flat ** 2, axis=2)
    e2 = jnp.sum(embed ** 2, axis=2)

    xv = x.reshape(B, D, M)
    emb = embed.reshape(K, D)
    em2 = (-2.0 * emb).astype(jnp.bfloat16)
    ebits = jax.lax.bitcast_convert_type(emb, jnp.uint32)
    ehi32 = jax.lax.bitcast_convert_type(
        ebits & jnp.uint32(0xFFFF0000), jnp.float32)
    ehi = ehi32.astype(jnp.bfloat16)
    elo = ((emb - ehi32) * 256.0).astype(jnp.bfloat16)
    ecat = jnp.concatenate([ehi, elo], axis=1)
    x2r = x2.reshape(B, 1, M)
    e2c = e2.reshape(K, 1)

    out_shape = [
        jax.ShapeDtypeStruct((B, D, M), jnp.float32),
        jax.ShapeDtypeStruct((B, 1, M), jnp.int32),
        jax.ShapeDtypeStruct((1, 1), jnp.float32),
        jax.ShapeDtypeStruct((1, 1), jnp.float32),
    ]
    UB = 2
    grid = (B // UB,)
    q, ind, diffo, perpo = pl.pallas_call(
        _vq_body,
        grid=grid,
        in_specs=[
            pl.BlockSpec((UB, D, M), lambda b: (b, 0, 0)),
            pl.BlockSpec((K, D), lambda b: (0, 0)),
            pl.BlockSpec((K, 2 * D), lambda b: (0, 0)),
            pl.BlockSpec((K, 1), lambda b: (0, 0)),
            pl.BlockSpec((UB, 1, M), lambda b: (b, 0, 0)),
        ],
        out_specs=[
            pl.BlockSpec((UB, D, M), lambda b: (b, 0, 0)),
            pl.BlockSpec((UB, 1, M), lambda b: (b, 0, 0)),
            pl.BlockSpec((1, 1), lambda b: (0, 0)),
            pl.BlockSpec((1, 1), lambda b: (0, 0)),
        ],
        out_shape=out_shape,
        scratch_shapes=[
            pltpu.VMEM((1, K), jnp.float32),
            pltpu.VMEM((1, 1), jnp.float32),
        ],
    )(xv, em2, ecat, e2c, x2r)

    quantized = q.reshape(B, C, H, W)
    embed_ind = ind.reshape(B, N, H, W)
    diff = diffo.reshape(())
    perplexity = perpo.reshape(N)
    return (quantized, diff, embed_ind, perplexity)
